# baseline (speedup 1.0000x reference)
import functools

import jax
import jax.numpy as jnp
from jax import lax
from jax.experimental import pallas as pl
from jax.experimental.pallas import tpu as pltpu
from jax.experimental.pallas import tpu_sc as plsc

_NC = 2
_NS = 16
_NW = _NC * _NS
_CHUNK = 128


@functools.lru_cache(maxsize=None)
def _make_gather(B, D):
    b_per_w = B // _NW
    nchunks = b_per_w // _CHUNK
    mesh = plsc.VectorSubcoreMesh(
        core_axis_name="c", subcore_axis_name="s",
        num_cores=_NC, num_subcores=_NS)

    @functools.partial(
        pl.kernel,
        out_type=(jax.ShapeDtypeStruct((B, D), jnp.float32),
                  jax.ShapeDtypeStruct((B, D), jnp.float32)),
        mesh=mesh,
        scratch_types=[
            pltpu.VMEM((b_per_w,), jnp.int32),
            pltpu.VMEM((b_per_w,), jnp.int32),
            pltpu.VMEM((_CHUNK, D), jnp.float32),
            pltpu.VMEM((_CHUNK, D), jnp.float32),
            pltpu.SemaphoreType.DMA,
        ],
        compiler_params=pltpu.CompilerParams(use_tc_tiling_on_sc=False),
    )
    def gather(a_hbm, b_hbm, ea_hbm, eb_hbm, xa_hbm, xb_hbm,
               ia_v, ib_v, ra_v, rb_v, sem):
        wid = lax.axis_index("s") * _NC + lax.axis_index("c")
        base = wid * b_per_w
        pltpu.sync_copy(a_hbm.at[pl.ds(base, b_per_w)], ia_v)
        pltpu.sync_copy(b_hbm.at[pl.ds(base, b_per_w)], ib_v)
        for t in range(nchunks):
            ca = pltpu.async_copy(
                ea_hbm.at[ia_v.at[pl.ds(t * _CHUNK, _CHUNK)]], ra_v, sem)
            cb = pltpu.async_copy(
                eb_hbm.at[ib_v.at[pl.ds(t * _CHUNK, _CHUNK)]], rb_v, sem)
            ca.wait()
            cb.wait()
            pltpu.sync_copy(ra_v, xa_hbm.at[pl.ds(base + t * _CHUNK, _CHUNK)])
            pltpu.sync_copy(rb_v, xb_hbm.at[pl.ds(base + t * _CHUNK, _CHUNK)])

    return gather


def _mlp_body(xa_ref, xb_ref, w1t_ref, b1_ref, w2t_ref, b2_ref, out_ref):
    x = xa_ref[...] + xb_ref[...]
    h = lax.dot_general(x, w1t_ref[...], (((1,), (0,)), ((), ())),
                        preferred_element_type=jnp.float32)
    h = jnp.maximum(h + b1_ref[...], 0.0)
    out = lax.dot_general(h, w2t_ref[...], (((1,), (0,)), ((), ())),
                          preferred_element_type=jnp.float32)
    out_ref[...] = out + b2_ref[...]


@functools.lru_cache(maxsize=None)
def _make_mlp(B, D, H, P, BT=512, PT=2048):
    nb = B // BT
    npt = pl.cdiv(P, PT)
    grid = (npt, nb)
    return pl.pallas_call(
        _mlp_body,
        grid=grid,
        in_specs=[
            pl.BlockSpec((BT, D), lambda j, i: (i, 0)),
            pl.BlockSpec((BT, D), lambda j, i: (i, 0)),
            pl.BlockSpec((D, H), lambda j, i: (0, 0)),
            pl.BlockSpec((1, H), lambda j, i: (0, 0)),
            pl.BlockSpec((H, PT), lambda j, i: (0, j)),
            pl.BlockSpec((1, PT), lambda j, i: (0, j)),
        ],
        out_specs=pl.BlockSpec((BT, PT), lambda j, i: (i, j)),
        out_shape=jax.ShapeDtypeStruct((B, P), jnp.float32),
        compiler_params=pltpu.CompilerParams(
            dimension_semantics=("arbitrary", "arbitrary")),
    )


def kernel(a, b, emb_a, emb_b, W1, b1, W2, b2):
    B = a.shape[0]
    P, D = emb_a.shape
    H = W1.shape[0]
    xa, xb = _make_gather(B, D)(a, b, emb_a, emb_b)
    mlp = _make_mlp(B, D, H, P)
    return mlp(xa, xb, W1.T, b1.reshape(1, H), W2.T, b2.reshape(1, P))


# speedup vs baseline: 1.0153x; 1.0153x over previous
import functools

import jax
import jax.numpy as jnp
from jax import lax
from jax.experimental import pallas as pl
from jax.experimental.pallas import tpu as pltpu
from jax.experimental.pallas import tpu_sc as plsc

_NC = 2
_NS = 16
_NW = _NC * _NS
_CHUNK = 128


@functools.lru_cache(maxsize=None)
def _make_gather(B, D):
    b_per_w = B // _NW
    nchunks = b_per_w // _CHUNK
    mesh = plsc.VectorSubcoreMesh(
        core_axis_name="c", subcore_axis_name="s",
        num_cores=_NC, num_subcores=_NS)

    @functools.partial(
        pl.kernel,
        out_type=(jax.ShapeDtypeStruct((B, D), jnp.float32),
                  jax.ShapeDtypeStruct((B, D), jnp.float32)),
        mesh=mesh,
        scratch_types=[
            pltpu.VMEM((b_per_w,), jnp.int32),
            pltpu.VMEM((b_per_w,), jnp.int32),
            pltpu.VMEM((_CHUNK, D), jnp.float32),
            pltpu.VMEM((_CHUNK, D), jnp.float32),
            pltpu.SemaphoreType.DMA,
        ],
        compiler_params=pltpu.CompilerParams(use_tc_tiling_on_sc=False),
    )
    def gather(a_hbm, b_hbm, ea_hbm, eb_hbm, xa_hbm, xb_hbm,
               ia_v, ib_v, ra_v, rb_v, sem):
        wid = lax.axis_index("s") * _NC + lax.axis_index("c")
        base = wid * b_per_w
        pltpu.sync_copy(a_hbm.at[pl.ds(base, b_per_w)], ia_v)
        pltpu.sync_copy(b_hbm.at[pl.ds(base, b_per_w)], ib_v)
        for t in range(nchunks):
            ca = pltpu.async_copy(
                ea_hbm.at[ia_v.at[pl.ds(t * _CHUNK, _CHUNK)]], ra_v, sem)
            cb = pltpu.async_copy(
                eb_hbm.at[ib_v.at[pl.ds(t * _CHUNK, _CHUNK)]], rb_v, sem)
            ca.wait()
            cb.wait()
            pltpu.sync_copy(ra_v, xa_hbm.at[pl.ds(base + t * _CHUNK, _CHUNK)])
            pltpu.sync_copy(rb_v, xb_hbm.at[pl.ds(base + t * _CHUNK, _CHUNK)])

    return gather


def _mlp_body(xa_ref, xb_ref, w1t_ref, b1_ref, w2_ref, b2_ref, out_ref,
              w2t_scr):
    @pl.when(pl.program_id(1) == 0)
    def _():
        w2t_scr[...] = w2_ref[...].T

    x = xa_ref[...] + xb_ref[...]
    h = lax.dot_general(x, w1t_ref[...], (((1,), (0,)), ((), ())),
                        preferred_element_type=jnp.float32)
    h = jnp.maximum(h + b1_ref[...], 0.0)
    out = lax.dot_general(h, w2t_scr[...], (((1,), (0,)), ((), ())),
                          preferred_element_type=jnp.float32)
    out_ref[...] = out + b2_ref[...]


@functools.lru_cache(maxsize=None)
def _make_mlp(B, D, H, P, BT=512, PT=2048):
    nb = B // BT
    npt = pl.cdiv(P, PT)
    grid = (npt, nb)
    return pl.pallas_call(
        _mlp_body,
        grid=grid,
        in_specs=[
            pl.BlockSpec((BT, D), lambda j, i: (i, 0)),
            pl.BlockSpec((BT, D), lambda j, i: (i, 0)),
            pl.BlockSpec((D, H), lambda j, i: (0, 0)),
            pl.BlockSpec((1, H), lambda j, i: (0, 0)),
            pl.BlockSpec((PT, H), lambda j, i: (j, 0)),
            pl.BlockSpec((1, PT), lambda j, i: (0, j)),
        ],
        out_specs=pl.BlockSpec((BT, PT), lambda j, i: (i, j)),
        out_shape=jax.ShapeDtypeStruct((B, P), jnp.float32),
        scratch_shapes=[pltpu.VMEM((H, PT), jnp.float32)],
        compiler_params=pltpu.CompilerParams(
            dimension_semantics=("arbitrary", "arbitrary")),
    )


def kernel(a, b, emb_a, emb_b, W1, b1, W2, b2):
    B = a.shape[0]
    P, D = emb_a.shape
    H = W1.shape[0]
    xa, xb = _make_gather(B, D)(a, b, emb_a, emb_b)
    mlp = _make_mlp(B, D, H, P)
    return mlp(xa, xb, W1.T, b1.reshape(1, H), W2, b2.reshape(1, P))


# speedup vs baseline: 2.9277x; 2.8836x over previous
import functools

import jax
import jax.numpy as jnp
from jax import lax
from jax.experimental import pallas as pl
from jax.experimental.pallas import tpu as pltpu
from jax.experimental.pallas import tpu_sc as plsc

_NC = 2
_NS = 16
_NW = _NC * _NS
_CHUNK = 128


@functools.lru_cache(maxsize=None)
def _make_gather(B, D):
    b_per_w = B // _NW
    nchunks = b_per_w // _CHUNK
    mesh = plsc.VectorSubcoreMesh(
        core_axis_name="c", subcore_axis_name="s",
        num_cores=_NC, num_subcores=_NS)

    @functools.partial(
        pl.kernel,
        out_type=(jax.ShapeDtypeStruct((B, D), jnp.float32),
                  jax.ShapeDtypeStruct((B, D), jnp.float32)),
        mesh=mesh,
        scratch_types=[
            pltpu.VMEM((b_per_w,), jnp.int32),
            pltpu.VMEM((b_per_w,), jnp.int32),
            pltpu.VMEM((_CHUNK, D), jnp.float32),
            pltpu.VMEM((_CHUNK, D), jnp.float32),
            pltpu.SemaphoreType.DMA,
        ],
        compiler_params=pltpu.CompilerParams(use_tc_tiling_on_sc=False),
    )
    def gather(a_hbm, b_hbm, ea_hbm, eb_hbm, xa_hbm, xb_hbm,
               ia_v, ib_v, ra_v, rb_v, sem):
        wid = lax.axis_index("s") * _NC + lax.axis_index("c")
        base = wid * b_per_w
        pltpu.sync_copy(a_hbm.at[pl.ds(base, b_per_w)], ia_v)
        pltpu.sync_copy(b_hbm.at[pl.ds(base, b_per_w)], ib_v)
        for t in range(nchunks):
            ca = pltpu.async_copy(
                ea_hbm.at[ia_v.at[pl.ds(t * _CHUNK, _CHUNK)]], ra_v, sem)
            cb = pltpu.async_copy(
                eb_hbm.at[ib_v.at[pl.ds(t * _CHUNK, _CHUNK)]], rb_v, sem)
            ca.wait()
            cb.wait()
            pltpu.sync_copy(ra_v, xa_hbm.at[pl.ds(base + t * _CHUNK, _CHUNK)])
            pltpu.sync_copy(rb_v, xb_hbm.at[pl.ds(base + t * _CHUNK, _CHUNK)])

    return gather


def _mlp_body(xa_ref, xb_ref, w1_ref, b1_ref, w2_ref, b2_ref, out_ref):
    x = xa_ref[...] + xb_ref[...]
    ht = lax.dot_general(w1_ref[...], x, (((1,), (1,)), ((), ())),
                         preferred_element_type=jnp.float32)
    ht = jnp.maximum(ht + b1_ref[...], 0.0)
    out = lax.dot_general(w2_ref[...], ht, (((1,), (0,)), ((), ())),
                          preferred_element_type=jnp.float32)
    out_ref[...] = out + b2_ref[...]


@functools.lru_cache(maxsize=None)
def _make_mlp(B, D, H, P, BT=512, PT=2048):
    nb = B // BT
    npt = pl.cdiv(P, PT)
    grid = (npt, nb)
    return pl.pallas_call(
        _mlp_body,
        grid=grid,
        in_specs=[
            pl.BlockSpec((BT, D), lambda j, i: (i, 0)),
            pl.BlockSpec((BT, D), lambda j, i: (i, 0)),
            pl.BlockSpec((H, D), lambda j, i: (0, 0)),
            pl.BlockSpec((H, 1), lambda j, i: (0, 0)),
            pl.BlockSpec((PT, H), lambda j, i: (j, 0)),
            pl.BlockSpec((PT, 1), lambda j, i: (j, 0)),
        ],
        out_specs=pl.BlockSpec((PT, BT), lambda j, i: (j, i)),
        out_shape=jax.ShapeDtypeStruct((P, B), jnp.float32),
        compiler_params=pltpu.CompilerParams(
            dimension_semantics=("arbitrary", "arbitrary")),
    )


def kernel(a, b, emb_a, emb_b, W1, b1, W2, b2):
    B = a.shape[0]
    P, D = emb_a.shape
    H = W1.shape[0]
    xa, xb = _make_gather(B, D)(a, b, emb_a, emb_b)
    mlp = _make_mlp(B, D, H, P)
    out_t = mlp(xa, xb, W1, b1.reshape(H, 1), W2, b2.reshape(P, 1))
    return out_t.T


# speedup vs baseline: 3.9276x; 1.3415x over previous
import functools

import jax
import jax.numpy as jnp
from jax import lax
from jax.experimental import pallas as pl
from jax.experimental.pallas import tpu as pltpu
from jax.experimental.pallas import tpu_sc as plsc

_NC = 2
_NS = 16
_NW = _NC * _NS
_CHUNK = 128


@functools.lru_cache(maxsize=None)
def _make_gather(B, D):
    b_per_w = B // _NW
    nchunks = b_per_w // _CHUNK
    mesh = plsc.VectorSubcoreMesh(
        core_axis_name="c", subcore_axis_name="s",
        num_cores=_NC, num_subcores=_NS)

    @functools.partial(
        pl.kernel,
        out_type=(jax.ShapeDtypeStruct((B, D), jnp.float32),
                  jax.ShapeDtypeStruct((B, D), jnp.float32)),
        mesh=mesh,
        scratch_types=[
            pltpu.VMEM((b_per_w,), jnp.int32),
            pltpu.VMEM((b_per_w,), jnp.int32),
            pltpu.VMEM((_CHUNK, D), jnp.float32),
            pltpu.VMEM((_CHUNK, D), jnp.float32),
            pltpu.SemaphoreType.DMA,
        ],
        compiler_params=pltpu.CompilerParams(use_tc_tiling_on_sc=False),
    )
    def gather(a_hbm, b_hbm, ea_hbm, eb_hbm, xa_hbm, xb_hbm,
               ia_v, ib_v, ra_v, rb_v, sem):
        wid = lax.axis_index("s") * _NC + lax.axis_index("c")
        base = wid * b_per_w
        pltpu.sync_copy(a_hbm.at[pl.ds(base, b_per_w)], ia_v)
        pltpu.sync_copy(b_hbm.at[pl.ds(base, b_per_w)], ib_v)
        for t in range(nchunks):
            ca = pltpu.async_copy(
                ea_hbm.at[ia_v.at[pl.ds(t * _CHUNK, _CHUNK)]], ra_v, sem)
            cb = pltpu.async_copy(
                eb_hbm.at[ib_v.at[pl.ds(t * _CHUNK, _CHUNK)]], rb_v, sem)
            ca.wait()
            cb.wait()
            pltpu.sync_copy(ra_v, xa_hbm.at[pl.ds(base + t * _CHUNK, _CHUNK)])
            pltpu.sync_copy(rb_v, xb_hbm.at[pl.ds(base + t * _CHUNK, _CHUNK)])

    return gather


def _ht_body(xa_ref, xb_ref, w1_ref, b1_ref, ht_ref):
    x = xa_ref[...] + xb_ref[...]
    ht = lax.dot_general(w1_ref[...], x, (((1,), (1,)), ((), ())),
                         preferred_element_type=jnp.float32)
    ht_ref[...] = jnp.maximum(ht + b1_ref[...], 0.0).astype(jnp.bfloat16)


@functools.lru_cache(maxsize=None)
def _make_ht(B, D, H, BT=2048):
    return pl.pallas_call(
        _ht_body,
        grid=(B // BT,),
        in_specs=[
            pl.BlockSpec((BT, D), lambda i: (i, 0)),
            pl.BlockSpec((BT, D), lambda i: (i, 0)),
            pl.BlockSpec((H, D), lambda i: (0, 0)),
            pl.BlockSpec((H, 1), lambda i: (0, 0)),
        ],
        out_specs=pl.BlockSpec((H, BT), lambda i: (0, i)),
        out_shape=jax.ShapeDtypeStruct((H, B), jnp.bfloat16),
    )


def _mlp_body(ht_ref, w2_ref, b2_ref, out_ref):
    w2 = w2_ref[...].astype(jnp.bfloat16)
    out = lax.dot_general(w2, ht_ref[...], (((1,), (0,)), ((), ())),
                          preferred_element_type=jnp.float32)
    out_ref[...] = out + b2_ref[...]


@functools.lru_cache(maxsize=None)
def _make_mlp(B, H, P, PT=256):
    npt = pl.cdiv(P, PT)
    return pl.pallas_call(
        _mlp_body,
        grid=(npt,),
        in_specs=[
            pl.BlockSpec((H, B), lambda j: (0, 0)),
            pl.BlockSpec((PT, H), lambda j: (j, 0)),
            pl.BlockSpec((PT, 1), lambda j: (j, 0)),
        ],
        out_specs=pl.BlockSpec((PT, B), lambda j: (j, 0)),
        out_shape=jax.ShapeDtypeStruct((P, B), jnp.float32),
        compiler_params=pltpu.CompilerParams(
            dimension_semantics=("arbitrary",)),
    )


def kernel(a, b, emb_a, emb_b, W1, b1, W2, b2):
    B = a.shape[0]
    P, D = emb_a.shape
    H = W1.shape[0]
    xa, xb = _make_gather(B, D)(a, b, emb_a, emb_b)
    ht = _make_ht(B, D, H)(xa, xb, W1, b1.reshape(H, 1))
    out_t = _make_mlp(B, H, P)(ht, W2, b2.reshape(P, 1))
    return out_t.T


# speedup vs baseline: 3.9598x; 1.0082x over previous
import functools

import jax
import jax.numpy as jnp
from jax import lax
from jax.experimental import pallas as pl
from jax.experimental.pallas import tpu as pltpu
from jax.experimental.pallas import tpu_sc as plsc

_NC = 2
_NS = 16
_NW = _NC * _NS
_CHUNK = 128


@functools.lru_cache(maxsize=None)
def _make_gather(B, D):
    b_per_w = B // _NW
    nchunks = b_per_w // _CHUNK
    mesh = plsc.VectorSubcoreMesh(
        core_axis_name="c", subcore_axis_name="s",
        num_cores=_NC, num_subcores=_NS)

    @functools.partial(
        pl.kernel,
        out_type=(jax.ShapeDtypeStruct((_NW, nchunks, _CHUNK, D),
                                       jnp.float32),
                  jax.ShapeDtypeStruct((_NW, nchunks, _CHUNK, D),
                                       jnp.float32)),
        mesh=mesh,
        scratch_types=[
            pltpu.VMEM((b_per_w,), jnp.int32),
            pltpu.VMEM((b_per_w,), jnp.int32),
            pltpu.VMEM((nchunks, _CHUNK, D), jnp.float32),
            pltpu.VMEM((nchunks, _CHUNK, D), jnp.float32),
            pltpu.SemaphoreType.DMA,
        ],
        compiler_params=pltpu.CompilerParams(use_tc_tiling_on_sc=False),
    )
    def gather(a_hbm, b_hbm, ea_hbm, eb_hbm, xa_hbm, xb_hbm,
               ia_v, ib_v, ra_v, rb_v, sem):
        wid = lax.axis_index("s") * _NC + lax.axis_index("c")
        base = wid * b_per_w
        pltpu.sync_copy(a_hbm.at[pl.ds(base, b_per_w)], ia_v)
        pltpu.sync_copy(b_hbm.at[pl.ds(base, b_per_w)], ib_v)
        copies = []
        for t in range(nchunks):
            copies.append(pltpu.async_copy(
                ea_hbm.at[ia_v.at[pl.ds(t * _CHUNK, _CHUNK)]],
                ra_v.at[t], sem))
            copies.append(pltpu.async_copy(
                eb_hbm.at[ib_v.at[pl.ds(t * _CHUNK, _CHUNK)]],
                rb_v.at[t], sem))
        for c in copies:
            c.wait()
        pltpu.sync_copy(ra_v, xa_hbm.at[wid])
        pltpu.sync_copy(rb_v, xb_hbm.at[wid])

    return gather


def _ht_body(xa_ref, xb_ref, w1_ref, b1_ref, ht_ref):
    x = xa_ref[...] + xb_ref[...]
    ht = lax.dot_general(w1_ref[...], x, (((1,), (1,)), ((), ())),
                         preferred_element_type=jnp.float32)
    ht_ref[...] = jnp.maximum(ht + b1_ref[...], 0.0).astype(jnp.bfloat16)


@functools.lru_cache(maxsize=None)
def _make_ht(B, D, H, BT=2048):
    return pl.pallas_call(
        _ht_body,
        grid=(B // BT,),
        in_specs=[
            pl.BlockSpec((BT, D), lambda i: (i, 0)),
            pl.BlockSpec((BT, D), lambda i: (i, 0)),
            pl.BlockSpec((H, D), lambda i: (0, 0)),
            pl.BlockSpec((H, 1), lambda i: (0, 0)),
        ],
        out_specs=pl.BlockSpec((H, BT), lambda i: (0, i)),
        out_shape=jax.ShapeDtypeStruct((H, B), jnp.bfloat16),
    )


def _mlp_body(ht_ref, w2_ref, b2_ref, out_ref):
    w2 = w2_ref[...].astype(jnp.bfloat16)
    out = lax.dot_general(w2, ht_ref[...], (((1,), (0,)), ((), ())),
                          preferred_element_type=jnp.float32)
    out_ref[...] = out + b2_ref[...]


@functools.lru_cache(maxsize=None)
def _make_mlp(B, H, P, PT=256):
    npt = pl.cdiv(P, PT)
    return pl.pallas_call(
        _mlp_body,
        grid=(npt,),
        in_specs=[
            pl.BlockSpec((H, B), lambda j: (0, 0)),
            pl.BlockSpec((PT, H), lambda j: (j, 0)),
            pl.BlockSpec((PT, 1), lambda j: (j, 0)),
        ],
        out_specs=pl.BlockSpec((PT, B), lambda j: (j, 0)),
        out_shape=jax.ShapeDtypeStruct((P, B), jnp.float32),
        compiler_params=pltpu.CompilerParams(
            dimension_semantics=("arbitrary",)),
    )


def kernel(a, b, emb_a, emb_b, W1, b1, W2, b2):
    B = a.shape[0]
    P, D = emb_a.shape
    H = W1.shape[0]
    xa, xb = _make_gather(B, D)(a, b, emb_a, emb_b)
    xa = xa.reshape(B, D)
    xb = xb.reshape(B, D)
    ht = _make_ht(B, D, H)(xa, xb, W1, b1.reshape(H, 1))
    out_t = _make_mlp(B, H, P)(ht, W2, b2.reshape(P, 1))
    return out_t.T
